# baseline (speedup 1.0000x reference)
import functools

import jax
import jax.numpy as jnp
from jax import lax
from jax.experimental import pallas as pl
from jax.experimental.pallas import tpu as pltpu
from jax.experimental.pallas import tpu_sc as plsc

B_SEG = 2048
NC, NS = 2, 16
NW = NC * NS
SEG_PER_W = B_SEG // NW
CHUNK = 512
LANES = 16
NVEC = 8
NEG_INF = float("-inf")


def _sc_segment_reduce(x, m32, edges):
    N, D = x.shape
    mesh = plsc.VectorSubcoreMesh(core_axis_name="c", subcore_axis_name="s")

    @functools.partial(
        pl.kernel,
        out_type=(
            jax.ShapeDtypeStruct((B_SEG, D), jnp.float32),
            jax.ShapeDtypeStruct((B_SEG, D), jnp.float32),
        ),
        mesh=mesh,
        scratch_types=[
            pltpu.VMEM((CHUNK, D), jnp.float32),
            pltpu.VMEM((CHUNK + LANES,), jnp.int32),
            pltpu.VMEM((SEG_PER_W, D), jnp.float32),
            pltpu.VMEM((SEG_PER_W, D), jnp.float32),
            pltpu.VMEM((SEG_PER_W + 2 * LANES,), jnp.int32),
        ],
    )
    def seg_kernel(x_hbm, m_hbm, edges_hbm, sum_hbm, max_hbm,
                   xbuf, mbuf, acc_s, acc_m, e_v):
        w = lax.axis_index("s") * NC + lax.axis_index("c")
        seg_lo = w * SEG_PER_W
        pltpu.sync_copy(edges_hbm.at[pl.ds(seg_lo, SEG_PER_W + 2 * LANES)], e_v)
        r0 = e_v[pl.ds(0, LANES)][0]
        r1 = e_v[pl.ds(SEG_PER_W, LANES)][0]

        zeros = jnp.zeros((LANES,), jnp.float32)
        ninf = jnp.full((LANES,), NEG_INF, jnp.float32)

        def init_body(i, _):
            s = i // NVEC
            j = i % NVEC
            acc_s[s, pl.ds(j * LANES, LANES)] = zeros
            acc_m[s, pl.ds(j * LANES, LANES)] = ninf
            return 0

        lax.fori_loop(0, SEG_PER_W * NVEC, init_body, 0)

        a0 = (r0 // 8) * 8
        nchunks = (r1 - a0 + CHUNK - 1) // CHUNK

        @pl.loop(0, nchunks, init_carry=jnp.int32(0))
        def _chunks(k, cur):
            s_un = a0 + k * CHUNK
            c0 = jnp.minimum(s_un, N - CHUNK)
            pltpu.sync_copy(x_hbm.at[pl.ds(c0, CHUNK)], xbuf)
            pltpu.sync_copy(m_hbm.at[pl.ds(c0, CHUNK)], mbuf.at[pl.ds(0, CHUNK)])
            lo = jnp.maximum(r0, s_un) - c0
            hi = jnp.minimum(r1, s_un + CHUNK) - c0

            hi0 = jnp.maximum(hi - 1, 0)
            m_last = mbuf[pl.ds(hi0, LANES)][0] - seg_lo
            m_last = jnp.minimum(jnp.maximum(m_last, cur - 1),
                                 SEG_PER_W - 1)

            @pl.loop(cur, m_last + 1)
            def _runs(si):
                st = jnp.maximum(e_v[pl.ds(si, LANES)][0] - c0, lo)
                en = jnp.minimum(e_v[pl.ds(si + 1, LANES)][0] - c0, hi)

                accs = tuple(acc_s[si, pl.ds(j * LANES, LANES)]
                             for j in range(NVEC))
                accm = tuple(acc_m[si, pl.ds(j * LANES, LANES)]
                             for j in range(NVEC))

                @pl.loop(st, en, init_carry=accs + accm)
                def out(r, carry):
                    a = carry[:NVEC]
                    m = carry[NVEC:]
                    vs = tuple(xbuf[r, pl.ds(j * LANES, LANES)]
                               for j in range(NVEC))
                    a = tuple(a[j] + vs[j] for j in range(NVEC))
                    m = tuple(jnp.maximum(m[j], vs[j]) for j in range(NVEC))
                    return a + m

                for j in range(NVEC):
                    acc_s[si, pl.ds(j * LANES, LANES)] = out[j]
                    acc_m[si, pl.ds(j * LANES, LANES)] = out[NVEC + j]

            return jnp.maximum(m_last, cur)

        pltpu.sync_copy(acc_s, sum_hbm.at[pl.ds(seg_lo, SEG_PER_W)])
        pltpu.sync_copy(acc_m, max_hbm.at[pl.ds(seg_lo, SEG_PER_W)])

    return seg_kernel(x, m32, edges)


def _tc_merge(seg_max, seg_sum, W_merge, b_merge):
    B, D = seg_max.shape

    def body(mx_ref, sm_ref, w_ref, b_ref, o_ref):
        mx = mx_ref[...]
        mx = jnp.where(jnp.isfinite(mx), mx, 0.0)
        acc = jnp.dot(mx, w_ref[0:D, :], preferred_element_type=jnp.float32)
        acc = acc + jnp.dot(sm_ref[...], w_ref[D:2 * D, :],
                            preferred_element_type=jnp.float32)
        o_ref[...] = acc + b_ref[...]

    return pl.pallas_call(
        body,
        out_shape=jax.ShapeDtypeStruct((B, W_merge.shape[1]), jnp.float32),
    )(seg_max, seg_sum, W_merge, b_merge)


def kernel(x, membership, W_merge, b_merge):
    m32 = membership.astype(jnp.int32)
    edges = jnp.searchsorted(
        m32, jnp.arange(B_SEG + 1, dtype=jnp.int32), side="left"
    ).astype(jnp.int32)
    edges = jnp.pad(edges, (0, 2 * LANES - 1), constant_values=2 ** 30)
    seg_sum, seg_max = _sc_segment_reduce(x, m32, edges)
    return _tc_merge(seg_max, seg_sum, W_merge, jnp.reshape(b_merge, (1, -1)))
